# initial kernel scaffold (unmeasured)
import jax
import jax.numpy as jnp
from jax import lax
from jax.experimental import pallas as pl
from jax.experimental.pallas import tpu as pltpu

N_DEV = 32
M_BLK = 128
K = 4096
N = 2048
K_BLK = 128


def kernel(x, w_mat):
    def body(x_ref, w_ref, out_ref, x_bf_ref, xg_ref, amax_ref,
             send_sems, recv_sems, amax_send_sems, amax_recv_sems):
        me = lax.axis_index("i")

        x_bf_ref[...] = x_ref[...].astype(jnp.bfloat16)

        data_sends = []
        for d in range(1, N_DEV):
            tgt = lax.rem(me + d, N_DEV)
            rdma = pltpu.make_async_remote_copy(
                src_ref=x_bf_ref.at[pl.ds(tgt * M_BLK, M_BLK), :],
                dst_ref=xg_ref.at[:, pl.ds(me * K_BLK, K_BLK)],
                send_sem=send_sems.at[d],
                recv_sem=recv_sems.at[me],
                device_id=(tgt,),
                device_id_type=pl.DeviceIdType.MESH,
            )
            rdma.start()
            data_sends.append(rdma)

        own = pltpu.make_async_copy(
            x_bf_ref.at[pl.ds(me * M_BLK, M_BLK), :],
            xg_ref.at[:, pl.ds(me * K_BLK, K_BLK)],
            send_sems.at[0],
        )
        own.start()

        w_bf = w_ref[...].astype(jnp.bfloat16)

        own.wait()
        for d in range(1, N_DEV):
            src = lax.rem(me + d, N_DEV)
            recv = pltpu.make_async_remote_copy(
                src_ref=x_bf_ref.at[pl.ds(0, M_BLK), :],
                dst_ref=xg_ref.at[:, pl.ds(src * K_BLK, K_BLK)],
                send_sem=send_sems.at[d],
                recv_sem=recv_sems.at[src],
                device_id=(src,),
                device_id_type=pl.DeviceIdType.MESH,
            )
            recv.wait_recv()

        y = jnp.dot(xg_ref[...], w_bf, preferred_element_type=jnp.float32)
        y = jnp.maximum(y, 0.0)
        local_amax = jnp.max(y)
        amax_ref[pl.ds(me, 1), :] = jnp.full((1, 128), local_amax,
                                             dtype=jnp.float32)

        amax_sends = []
        for d in range(1, N_DEV):
            tgt = lax.rem(me + d, N_DEV)
            rdma = pltpu.make_async_remote_copy(
                src_ref=amax_ref.at[pl.ds(me, 1), :],
                dst_ref=amax_ref.at[pl.ds(me, 1), :],
                send_sem=amax_send_sems.at[d],
                recv_sem=amax_recv_sems.at[me],
                device_id=(tgt,),
                device_id_type=pl.DeviceIdType.MESH,
            )
            rdma.start()
            amax_sends.append(rdma)
        for d in range(1, N_DEV):
            src = lax.rem(me + d, N_DEV)
            recv = pltpu.make_async_remote_copy(
                src_ref=amax_ref.at[pl.ds(src, 1), :],
                dst_ref=amax_ref.at[pl.ds(src, 1), :],
                send_sem=amax_send_sems.at[d],
                recv_sem=amax_recv_sems.at[src],
                device_id=(src,),
                device_id_type=pl.DeviceIdType.MESH,
            )
            recv.wait_recv()

        g_amax = jnp.max(amax_ref[...])

        inv = jnp.where(g_amax > 0, 448.0 / g_amax, 0.0)
        ys = jnp.minimum(y * inv, 448.0)
        q = ys.astype(jnp.float8_e4m3fn)
        out_ref[...] = q.astype(jnp.float32) * (g_amax / 448.0)

        for r in data_sends:
            r.wait_send()
        for r in amax_sends:
            r.wait_send()

    return pl.pallas_call(
        body,
        out_shape=jax.ShapeDtypeStruct((M_BLK, N), jnp.float32),
        in_specs=[
            pl.BlockSpec(memory_space=pltpu.VMEM),
            pl.BlockSpec(memory_space=pltpu.VMEM),
        ],
        out_specs=pl.BlockSpec(memory_space=pltpu.VMEM),
        scratch_shapes=[
            pltpu.VMEM((K, K_BLK), jnp.bfloat16),
            pltpu.VMEM((M_BLK, K), jnp.bfloat16),
            pltpu.VMEM((N_DEV, 128), jnp.float32),
            pltpu.SemaphoreType.DMA((N_DEV,)),
            pltpu.SemaphoreType.DMA((N_DEV,)),
            pltpu.SemaphoreType.DMA((N_DEV,)),
            pltpu.SemaphoreType.DMA((N_DEV,)),
        ],
        compiler_params=pltpu.CompilerParams(collective_id=0),
    )(x, w_mat)


# baseline (device time: 38559 ns/iter reference)
import jax
import jax.numpy as jnp
from jax import lax
from jax.experimental import pallas as pl
from jax.experimental.pallas import tpu as pltpu

N_DEV = 32
M_BLK = 128
K = 4096
N = 2048
K_BLK = 128
K_CH = 512
K_CHUNKS = K // K_CH
BLKS_PER_CH = K_CH // K_BLK
W_SLOTS = 6
W_PREFETCH = 4


def kernel(x, w_mat):
    def body(x_ref, w_ref, out_ref, x_bf_ref, xg_ref, amax_ref, w_buf,
             send_sems, recv_sems, amax_send_sems, amax_recv_sems, w_sems):
        me = lax.axis_index("i")

        barrier_sem = pltpu.get_barrier_semaphore()
        for d in range(1, N_DEV):
            pl.semaphore_signal(
                barrier_sem, inc=1,
                device_id=(lax.rem(me + d, N_DEV),),
                device_id_type=pl.DeviceIdType.MESH,
            )

        def w_dma(c):
            return pltpu.make_async_copy(
                w_ref.at[pl.ds(c * K_CH, K_CH), :],
                w_buf.at[c % W_SLOTS],
                w_sems.at[c % W_SLOTS],
            )

        for c in range(W_PREFETCH):
            w_dma(c).start()

        x_bf_ref[...] = x_ref[...].astype(jnp.bfloat16)

        own = pltpu.make_async_copy(
            x_bf_ref.at[pl.ds(me * M_BLK, M_BLK), :],
            xg_ref.at[:, pl.ds(me * K_BLK, K_BLK)],
            recv_sems.at[me],
        )
        own.start()

        pl.semaphore_wait(barrier_sem, N_DEV - 1)

        data_sends = []
        for d in range(1, N_DEV):
            tgt = lax.rem(me + d, N_DEV)
            rdma = pltpu.make_async_remote_copy(
                src_ref=x_bf_ref.at[pl.ds(tgt * M_BLK, M_BLK), :],
                dst_ref=xg_ref.at[:, pl.ds(me * K_BLK, K_BLK)],
                send_sem=send_sems.at[d],
                recv_sem=recv_sems.at[me],
                device_id=(tgt,),
                device_id_type=pl.DeviceIdType.MESH,
            )
            rdma.start()
            data_sends.append(rdma)

        y = None
        for c in range(K_CHUNKS):
            for s in range(c * BLKS_PER_CH, (c + 1) * BLKS_PER_CH):
                pltpu.make_async_remote_copy(
                    src_ref=x_bf_ref.at[pl.ds(0, M_BLK), :],
                    dst_ref=xg_ref.at[:, pl.ds(s * K_BLK, K_BLK)],
                    send_sem=send_sems.at[0],
                    recv_sem=recv_sems.at[s],
                    device_id=(0,),
                    device_id_type=pl.DeviceIdType.MESH,
                ).wait_recv()
            w_dma(c).wait()
            xc = xg_ref[:, c * K_CH:(c + 1) * K_CH].astype(jnp.float32)
            d = jnp.dot(xc, w_buf[c % W_SLOTS],
                        preferred_element_type=jnp.float32)
            y = d if y is None else y + d
            if c + W_PREFETCH < K_CHUNKS:
                w_dma(c + W_PREFETCH).start()

        local_amax = jnp.maximum(jnp.max(y), 0.0)
        amax_ref[pl.ds(me, 1), :] = jnp.full((1, 128), local_amax,
                                             dtype=jnp.float32)
        amax_sends = []
        for d in range(1, N_DEV):
            tgt = lax.rem(me + d, N_DEV)
            rdma = pltpu.make_async_remote_copy(
                src_ref=amax_ref.at[pl.ds(me, 1), :],
                dst_ref=amax_ref.at[pl.ds(me, 1), :],
                send_sem=amax_send_sems.at[d],
                recv_sem=amax_recv_sems.at[me],
                device_id=(tgt,),
                device_id_type=pl.DeviceIdType.MESH,
            )
            rdma.start()
            amax_sends.append(rdma)

        y = jnp.maximum(y, 0.0)

        for s in range(N_DEV):
            pass
        for d in range(1, N_DEV):
            src = lax.rem(me + d, N_DEV)
            pltpu.make_async_remote_copy(
                src_ref=amax_ref.at[pl.ds(0, 1), :],
                dst_ref=amax_ref.at[pl.ds(src, 1), :],
                send_sem=amax_send_sems.at[0],
                recv_sem=amax_recv_sems.at[src],
                device_id=(0,),
                device_id_type=pl.DeviceIdType.MESH,
            ).wait_recv()

        g_amax = jnp.max(amax_ref[...])

        inv = jnp.where(g_amax > 0, 448.0 / g_amax, 0.0)
        ys = jnp.minimum(y * inv, 448.0)
        q = ys.astype(jnp.float8_e4m3fn)
        out_ref[...] = q.astype(jnp.float32) * (g_amax / 448.0)

        for r in data_sends:
            r.wait_send()
        for r in amax_sends:
            r.wait_send()

    return pl.pallas_call(
        body,
        out_shape=jax.ShapeDtypeStruct((M_BLK, N), jnp.float32),
        in_specs=[
            pl.BlockSpec(memory_space=pltpu.VMEM),
            pl.BlockSpec(memory_space=pltpu.MemorySpace.HBM),
        ],
        out_specs=pl.BlockSpec(memory_space=pltpu.VMEM),
        scratch_shapes=[
            pltpu.VMEM((K, K_BLK), jnp.bfloat16),
            pltpu.VMEM((M_BLK, K), jnp.bfloat16),
            pltpu.VMEM((N_DEV, 128), jnp.float32),
            pltpu.VMEM((W_SLOTS, K_CH, N), jnp.float32),
            pltpu.SemaphoreType.DMA((N_DEV,)),
            pltpu.SemaphoreType.DMA((N_DEV,)),
            pltpu.SemaphoreType.DMA((N_DEV,)),
            pltpu.SemaphoreType.DMA((N_DEV,)),
            pltpu.SemaphoreType.DMA((W_SLOTS,)),
        ],
        compiler_params=pltpu.CompilerParams(collective_id=0),
    )(x, w_mat)
